# baseline (speedup 1.0000x reference)
import functools
import math

import jax
import jax.numpy as jnp
from jax import lax
from jax.experimental import pallas as pl
from jax.experimental.pallas import tpu as pltpu
from jax.experimental.pallas import tpu_sc as plsc

N = 10000
HIDDEN = 256
K = 16
HEADS = 8
HEAD_DIM = HIDDEN // HEADS
NORM = math.sqrt(1.0 / HEAD_DIM)

GROUP = 8
N_GROUPS = N // GROUP
N_WORKERS = 32
GROUPS_PER_WORKER = -(-N_GROUPS // N_WORKERS)
L = 16
D_UNROLL = 4

HALF_W = (K // 2) * HIDDEN
HOFF = HALF_W + 8
NBROW = 2 * HALF_W + 8 + 9
XROW = 257


def _treesum(xs):
    xs = list(xs)
    while len(xs) > 1:
        xs = [xs[i] + xs[i + 1] for i in range(0, len(xs) - 1, 2)] + (
            [xs[-1]] if len(xs) % 2 else [])
    return xs[0]


def _body(x_hbm, nb_hbm, out_hbm, xbuf, nbuf, obuf, isem0, isem1, osem0, osem1):
    wid = lax.axis_index("s") * 2 + lax.axis_index("c")
    isems = (isem0, isem1)
    osems = (osem0, osem1)

    lane = lax.iota(jnp.int32, L)
    half = lane >> 3
    node = jnp.where(half == 0, lane, 15 - lane)
    nodep = (node, node + GROUP)
    krow_off = [m * HIDDEN + HOFF * half for m in range(K // 2)]
    lo_mask = lane < 8
    norm_v = jnp.full((L,), NORM, jnp.float32)

    def swap_halves(v):
        return jnp.flip(v, axis=0)

    def in_copies(g, p):
        base = g * GROUP
        for j in range(GROUP):
            row = p * GROUP + j
            for hf in range(2):
                yield (nb_hbm.at[base + j, pl.ds(hf * HALF_W, HALF_W)],
                       nbuf.at[row, pl.ds(hf * HOFF, HALF_W)], isems[p])
            yield (x_hbm.at[base + j], xbuf.at[row, pl.ds(0, HIDDEN)],
                   isems[p])

    def out_copies(g, p):
        base = g * GROUP
        for j in range(GROUP):
            yield (obuf.at[p * GROUP + j, pl.ds(0, HIDDEN)],
                   out_hbm.at[base + j], osems[p])

    def fire(copies):
        for src, dst, sem in copies:
            pltpu.async_copy(src, dst, sem)

    def drain(copies):
        for src, dst, sem in copies:
            pltpu.make_async_copy(src, dst, sem).wait()

    def compute(g, p):
        nd = nodep[p]

        def head_body(h, _):
            off = h * HEAD_DIM
            offv = jnp.full((L,), off, jnp.int32)
            qb = offv
            kb = [b + offv for b in krow_off]

            def score_step(t, accs):
                accs = list(accs)
                d0v = jnp.full((L,), t * D_UNROLL, jnp.int32)
                for u in range(D_UNROLL):
                    duv = d0v + u if u else d0v
                    qv = plsc.load_gather(xbuf, [nd, qb + duv])
                    for m in range(K // 2):
                        kv = plsc.load_gather(nbuf, [nd, kb[m] + duv])
                        accs[m] = accs[m] + qv * kv
                return tuple(accs)

            zeros = tuple(jnp.zeros((L,), jnp.float32) for _ in range(K // 2))
            accs = lax.fori_loop(0, HEAD_DIM // D_UNROLL, score_step, zeros)

            scaled = [a * norm_v for a in accs]
            mx = scaled[0]
            for a in scaled[1:]:
                mx = jnp.maximum(mx, a)
            mx = jnp.maximum(mx, swap_halves(mx))
            es = [jnp.exp(a - mx) for a in scaled]
            s = _treesum(es)
            s = s + swap_halves(s)
            inv = 1.0 / s
            ws = tuple(e * inv for e in es)

            def out_step(t, carry):
                w = carry
                d0v = jnp.full((L,), t * D_UNROLL, jnp.int32)
                for u in range(D_UNROLL):
                    duv = d0v + u if u else d0v
                    prods = [w[m] * plsc.load_gather(nbuf, [nd, kb[m] + duv])
                             for m in range(K // 2)]
                    acc = _treesum(prods)
                    acc = acc + swap_halves(acc)
                    plsc.store_scatter(obuf, [nd, qb + duv], acc, mask=lo_mask)
                return carry

            lax.fori_loop(0, HEAD_DIM // D_UNROLL, out_step, ws)
            return 0

        lax.fori_loop(0, HEADS, head_body, 0)

    fire(in_copies(wid, 0))
    fire(in_copies(wid + N_WORKERS, 1))

    def pair_body(i2, _):
        for p in range(2):
            i = i2 * 2 + p
            g = wid + i * N_WORKERS

            def run():
                drain(in_copies(g, p))
                pl.when(i >= 2)(lambda: drain(out_copies(g - 2 * N_WORKERS, p)))
                compute(g, p)
                fire(out_copies(g, p))
                g2 = g + 2 * N_WORKERS
                pl.when(g2 < N_GROUPS)(lambda: fire(in_copies(g2, p)))

            pl.when(g < N_GROUPS)(run)
        return 0

    lax.fori_loop(0, GROUPS_PER_WORKER // 2, pair_body, 0)

    nv = (N_GROUPS - wid + N_WORKERS - 1) // N_WORKERS
    last = nv - 1
    for p in range(2):
        ip = jnp.where((last & 1) == p, last, last - 1)
        drain(out_copies(wid + ip * N_WORKERS, p))


_attn = functools.partial(
    pl.kernel,
    out_type=jax.ShapeDtypeStruct((N, HIDDEN), jnp.float32),
    mesh=plsc.VectorSubcoreMesh(core_axis_name="c", subcore_axis_name="s"),
    compiler_params=pltpu.CompilerParams(
        use_tc_tiling_on_sc=False, needs_layout_passes=False),
    scratch_types=[
        pltpu.VMEM((2 * GROUP, XROW), jnp.float32),
        pltpu.VMEM((2 * GROUP, NBROW), jnp.float32),
        pltpu.VMEM((2 * GROUP, XROW), jnp.float32),
        pltpu.SemaphoreType.DMA,
        pltpu.SemaphoreType.DMA,
        pltpu.SemaphoreType.DMA,
        pltpu.SemaphoreType.DMA,
    ],
)(_body)


def kernel(x, neighbors):
    return _attn(x, neighbors.reshape(N, K * HIDDEN))


# speedup vs baseline: 1.0373x; 1.0373x over previous
import functools
import math

import jax
import jax.numpy as jnp
from jax import lax
from jax.experimental import pallas as pl
from jax.experimental.pallas import tpu as pltpu
from jax.experimental.pallas import tpu_sc as plsc

N = 10000
HIDDEN = 256
K = 16
HEADS = 8
HEAD_DIM = HIDDEN // HEADS
NORM = math.sqrt(1.0 / HEAD_DIM)

GROUP = 8
N_GROUPS = N // GROUP
N_WORKERS = 32
GROUPS_PER_WORKER = -(-N_GROUPS // N_WORKERS)
L = 16
D_UNROLL = 4

HALF_W = (K // 2) * HIDDEN
HOFF = HALF_W + 8
NBROW = 2 * HALF_W + 8 + 9
XROW = 257


def _treesum(xs):
    xs = list(xs)
    while len(xs) > 1:
        xs = [xs[i] + xs[i + 1] for i in range(0, len(xs) - 1, 2)] + (
            [xs[-1]] if len(xs) % 2 else [])
    return xs[0]


def _body(x_hbm, nb_hbm, out_hbm, xbuf, nbuf, obuf, isem0, isem1, osem0, osem1):
    wid = lax.axis_index("s") * 2 + lax.axis_index("c")
    isems = (isem0, isem1)
    osems = (osem0, osem1)

    lane = lax.iota(jnp.int32, L)
    half = lane >> 3
    node = jnp.where(half == 0, lane, 15 - lane)
    nodep = (node, node + GROUP)
    krow_off = [m * HIDDEN + HOFF * half for m in range(K // 2)]
    lo_mask = lane < 8
    norm_v = jnp.full((L,), NORM, jnp.float32)

    def swap_halves(v):
        return jnp.flip(v, axis=0)

    def in_copies(g, p):
        base = g * GROUP
        for j in range(GROUP):
            row = p * GROUP + j
            for hf in range(2):
                yield (nb_hbm.at[base + j, pl.ds(hf * HALF_W, HALF_W)],
                       nbuf.at[row, pl.ds(hf * HOFF, HALF_W)], isems[p])
            yield (x_hbm.at[base + j], xbuf.at[row, pl.ds(0, HIDDEN)],
                   isems[p])

    def out_copies(g, p):
        base = g * GROUP
        for j in range(GROUP):
            yield (obuf.at[p * GROUP + j, pl.ds(0, HIDDEN)],
                   out_hbm.at[base + j], osems[p])

    def fire(copies):
        for src, dst, sem in copies:
            pltpu.async_copy(src, dst, sem)

    def drain(copies):
        for src, dst, sem in copies:
            pltpu.make_async_copy(src, dst, sem).wait()

    def compute(g, p):
        nd = nodep[p]

        def head_body(h, _):
            off = h * HEAD_DIM
            offv = jnp.full((L,), off, jnp.int32)
            qb = offv
            kb = [b + offv for b in krow_off]

            def score_step(t, accs):
                accs = list(accs)
                d0v = jnp.full((L,), t, jnp.int32)
                for u in range(D_UNROLL):
                    duv = d0v + u if u else d0v
                    qv = plsc.load_gather(xbuf, [nd, qb + duv])
                    for m in range(K // 2):
                        kv = plsc.load_gather(nbuf, [nd, kb[m] + duv])
                        accs[m] = accs[m] + qv * kv
                return tuple(accs)

            zeros = tuple(jnp.zeros((L,), jnp.float32) for _ in range(K // 2))
            accs = plsc.parallel_loop(
                0, HEAD_DIM, D_UNROLL, unroll=2, carry=zeros)(score_step)

            scaled = [a * norm_v for a in accs]
            mx = scaled[0]
            for a in scaled[1:]:
                mx = jnp.maximum(mx, a)
            mx = jnp.maximum(mx, swap_halves(mx))
            es = [jnp.exp(a - mx) for a in scaled]
            s = _treesum(es)
            s = s + swap_halves(s)
            inv = 1.0 / s
            ws = tuple(e * inv for e in es)

            def out_step(t, carry):
                w = carry
                d0v = jnp.full((L,), t, jnp.int32)
                for u in range(D_UNROLL):
                    duv = d0v + u if u else d0v
                    prods = [w[m] * plsc.load_gather(nbuf, [nd, kb[m] + duv])
                             for m in range(K // 2)]
                    acc = _treesum(prods)
                    acc = acc + swap_halves(acc)
                    plsc.store_scatter(obuf, [nd, qb + duv], acc, mask=lo_mask)
                return carry

            plsc.parallel_loop(0, HEAD_DIM, D_UNROLL, unroll=2, carry=ws)(out_step)
            return 0

        lax.fori_loop(0, HEADS, head_body, 0)

    fire(in_copies(wid, 0))
    fire(in_copies(wid + N_WORKERS, 1))

    def pair_body(i2, _):
        for p in range(2):
            i = i2 * 2 + p
            g = wid + i * N_WORKERS

            def run():
                drain(in_copies(g, p))
                pl.when(i >= 2)(lambda: drain(out_copies(g - 2 * N_WORKERS, p)))
                compute(g, p)
                fire(out_copies(g, p))
                g2 = g + 2 * N_WORKERS
                pl.when(g2 < N_GROUPS)(lambda: fire(in_copies(g2, p)))

            pl.when(g < N_GROUPS)(run)
        return 0

    lax.fori_loop(0, GROUPS_PER_WORKER // 2, pair_body, 0)

    nv = (N_GROUPS - wid + N_WORKERS - 1) // N_WORKERS
    last = nv - 1
    for p in range(2):
        ip = jnp.where((last & 1) == p, last, last - 1)
        drain(out_copies(wid + ip * N_WORKERS, p))


_attn = functools.partial(
    pl.kernel,
    out_type=jax.ShapeDtypeStruct((N, HIDDEN), jnp.float32),
    mesh=plsc.VectorSubcoreMesh(core_axis_name="c", subcore_axis_name="s"),
    compiler_params=pltpu.CompilerParams(
        use_tc_tiling_on_sc=False, needs_layout_passes=False),
    scratch_types=[
        pltpu.VMEM((2 * GROUP, XROW), jnp.float32),
        pltpu.VMEM((2 * GROUP, NBROW), jnp.float32),
        pltpu.VMEM((2 * GROUP, XROW), jnp.float32),
        pltpu.SemaphoreType.DMA,
        pltpu.SemaphoreType.DMA,
        pltpu.SemaphoreType.DMA,
        pltpu.SemaphoreType.DMA,
    ],
)(_body)


def kernel(x, neighbors):
    return _attn(x, neighbors.reshape(N, K * HIDDEN))
